# baseline (speedup 1.0000x reference)
import functools

import jax
import jax.numpy as jnp
from jax import lax
from jax.experimental import pallas as pl
from jax.experimental.pallas import tpu as pltpu
from jax.experimental.pallas import tpu_sc as plsc

VOCAB = 100000
EMBED_DIM = 128
CONTEXT = 200
HIDDEN = 128
KDIM = CONTEXT * EMBED_DIM

V_TILE = 5000
N_TILES = VOCAB // V_TILE

ROWS_PER_WORKER = 8
N_WORKERS = CONTEXT // ROWS_PER_WORKER


def _sc_gather_kernel(idx_hbm, table_hbm, out_hbm, idx_v, rows_v, sem):
    wid = lax.axis_index("s") * 2 + lax.axis_index("c")

    @pl.when(wid < N_WORKERS)
    def _():
        base = wid * ROWS_PER_WORKER
        pltpu.sync_copy(idx_hbm.at[pl.ds(base, ROWS_PER_WORKER)], idx_v)
        pltpu.async_copy(table_hbm.at[idx_v], rows_v, sem).wait()
        pltpu.sync_copy(rows_v, out_hbm.at[pl.ds(base, ROWS_PER_WORKER)])


@jax.jit
def _sc_gather(idx, table):
    mesh = plsc.VectorSubcoreMesh(core_axis_name="c", subcore_axis_name="s")
    return pl.kernel(
        _sc_gather_kernel,
        mesh=mesh,
        out_type=jax.ShapeDtypeStruct((CONTEXT, EMBED_DIM), jnp.float32),
        scratch_types=[
            pltpu.VMEM((ROWS_PER_WORKER,), jnp.int32),
            pltpu.VMEM((ROWS_PER_WORKER, EMBED_DIM), jnp.float32),
            pltpu.SemaphoreType.DMA,
        ],
    )(idx, table)


def _mlp_kernel(emb_ref, w1_ref, b1_ref, w2_ref, b2_ref, out_ref, hid_ref):
    t = pl.program_id(0)

    @pl.when(t == 0)
    def _():
        h = lax.dot_general(
            emb_ref[...], w1_ref[...], (((1,), (1,)), ((), ())),
            preferred_element_type=jnp.float32)
        hid_ref[...] = jnp.maximum(h + b1_ref[...], 0.0)

    logits = lax.dot_general(
        hid_ref[...], w2_ref[...], (((1,), (1,)), ((), ())),
        preferred_element_type=jnp.float32)
    out_ref[pl.ds(t, 1), :] = logits + b2_ref[pl.ds(t, 1), :]

    @pl.when(t == N_TILES - 1)
    def _():
        x = out_ref[...]
        m = jnp.max(x)
        lse = m + jnp.log(jnp.sum(jnp.exp(x - m)))
        out_ref[...] = x - lse


@jax.jit
def _tc_mlp(embflat, W1, b1, W2, b2):
    return pl.pallas_call(
        _mlp_kernel,
        grid=(N_TILES,),
        in_specs=[
            pl.BlockSpec((1, KDIM), lambda t: (0, 0)),
            pl.BlockSpec((HIDDEN, KDIM), lambda t: (0, 0)),
            pl.BlockSpec((1, HIDDEN), lambda t: (0, 0)),
            pl.BlockSpec((V_TILE, EMBED_DIM), lambda t: (t, 0)),
            pl.BlockSpec((N_TILES, V_TILE), lambda t: (0, 0)),
        ],
        out_specs=pl.BlockSpec((N_TILES, V_TILE), lambda t: (0, 0)),
        out_shape=jax.ShapeDtypeStruct((N_TILES, V_TILE), jnp.float32),
        scratch_shapes=[pltpu.VMEM((1, HIDDEN), jnp.float32)],
        compiler_params=pltpu.CompilerParams(
            dimension_semantics=("arbitrary",)),
    )(embflat, W1, b1, W2, b2)


def kernel(inputs, emb, W1, b1, W2, b2):
    embeds = _sc_gather(inputs.astype(jnp.int32), emb)
    embflat = embeds.reshape(1, KDIM)
    out = _tc_mlp(embflat, W1, b1.reshape(1, HIDDEN), W2,
                  b2.reshape(N_TILES, V_TILE))
    return out.reshape(1, VOCAB)
